# baseline (speedup 1.0000x reference)
import functools

import numpy as np
import jax
import jax.numpy as jnp
from jax import lax
from jax.experimental import pallas as pl
from jax.experimental.pallas import tpu as pltpu
from jax.experimental.pallas import tpu_sc as plsc

_MASK_INDEXES = (1, 4, 7, 10, 13)
_PPI = 64

_consts_cache = {}


def _rotl32(x, r):
    return ((x << np.uint32(r)) | (x >> np.uint32(32 - r))).astype(np.uint32)


def _threefry2x32(k0, k1, x0, x1):
    ks0 = np.uint32(k0)
    ks1 = np.uint32(k1)
    ks2 = np.uint32(ks0 ^ ks1 ^ np.uint32(0x1BD11BDA))
    x0 = (x0 + ks0).astype(np.uint32)
    x1 = (x1 + ks1).astype(np.uint32)
    rots = ((13, 15, 26, 6), (17, 29, 16, 24))
    ks = (ks0, ks1, ks2)
    for i in range(5):
        for r in rots[i % 2]:
            x0 = (x0 + x1).astype(np.uint32)
            x1 = _rotl32(x1, r)
            x1 = (x1 ^ x0).astype(np.uint32)
        x0 = (x0 + ks[(i + 1) % 3]).astype(np.uint32)
        x1 = (x1 + ks[(i + 2) % 3] + np.uint32(i + 1)).astype(np.uint32)
    return x0, x1


def _uniform_key42(shape):
    n = int(np.prod(shape))
    i64 = np.arange(n, dtype=np.uint64)
    c_hi = (i64 >> np.uint64(32)).astype(np.uint32)
    c_lo = (i64 & np.uint64(0xFFFFFFFF)).astype(np.uint32)
    b0, b1 = _threefry2x32(0, 42, c_hi, c_lo)
    bits = (b0 ^ b1).astype(np.uint32)
    f = ((bits >> np.uint32(9)) | np.uint32(0x3F800000)).view(np.float32)
    f = f - np.float32(1.0)
    return np.maximum(np.float32(0.0), f).reshape(shape)


def _constants(N, L):
    ck = (N, L)
    if ck not in _consts_cache:
        noise = _uniform_key42((N, L))
        masked_pos = np.array(
            [idx * _PPI + i for idx in _MASK_INDEXES for i in range(_PPI)],
            dtype=np.int64,
        )
        noise[:, masked_pos] = 2.0
        len_keep = L - masked_pos.size
        ids_shuffle = np.argsort(noise, axis=1, kind="stable").astype(np.int32)
        ids_restore = np.argsort(ids_shuffle, axis=1, kind="stable").astype(np.int32)
        ids_keep = ids_shuffle[:, :len_keep]
        mask = np.ones((N, L), dtype=np.float32)
        mask[:, :len_keep] = 0.0
        mask = np.take_along_axis(mask, ids_restore, axis=1)
        _consts_cache[ck] = (ids_keep, mask, ids_restore, len_keep)
    return _consts_cache[ck]


def _make_gather(N, L, D, K, NC, NS):
    NW = NC * NS
    SPW = N // NW
    DP = 32
    Q = D // DP
    T = SPW * Q
    JV = K // 16
    JH = 2
    JPH = JV // JH
    mesh = plsc.VectorSubcoreMesh(core_axis_name="c", subcore_axis_name="s")

    @functools.partial(
        pl.kernel,
        mesh=mesh,
        compiler_params=pltpu.CompilerParams(needs_layout_passes=False),
        out_type=jax.ShapeDtypeStruct((N * D * K,), jnp.float32),
        scratch_types=[
            pltpu.VMEM((SPW * K,), jnp.int32),
            pltpu.VMEM((DP * L,), jnp.float32),
            pltpu.VMEM((DP * L,), jnp.float32),
            pltpu.VMEM((DP * K,), jnp.float32),
            pltpu.VMEM((DP * K,), jnp.float32),
            pltpu.SemaphoreType.DMA,
            pltpu.SemaphoreType.DMA,
            pltpu.SemaphoreType.DMA,
            pltpu.SemaphoreType.DMA,
        ],
    )
    def gather_k(xt_hbm, idx_hbm, out_hbm, idx_v, sbuf0, sbuf1, obuf0, obuf1,
                 gsem0, gsem1, ssem0, ssem1):
        wid = lax.axis_index("s") * NC + lax.axis_index("c")
        sbufs = (sbuf0, sbuf1)
        obufs = (obuf0, obuf1)
        gsems = (gsem0, gsem1)
        ssems = (ssem0, ssem1)
        n0 = wid * SPW

        def src_slice(t):
            base = ((n0 + t // Q) * D + (t % Q) * DP) * L
            return xt_hbm.at[pl.ds(base, DP * L)]

        def out_slice(t):
            base = ((n0 + t // Q) * D + (t % Q) * DP) * K
            return out_hbm.at[pl.ds(base, DP * K)]

        pltpu.sync_copy(idx_hbm.at[pl.ds(n0 * K, SPW * K)], idx_v)

        def compute(t, src_v, out_v):
            nloc = t // Q
            for jh in range(JH):
                col0 = nloc * K + jh * JPH * 16
                regs = tuple(
                    idx_v[pl.ds(col0 + j * 16, 16)] for j in range(JPH)
                )

                def body(d, regs, jh=jh):
                    new = []
                    for j in range(JPH):
                        g = plsc.load_gather(src_v, [regs[j]])
                        out_v[pl.ds(d * K + jh * JPH * 16 + j * 16, 16)] = g
                        new.append(regs[j] + np.int32(L))
                    return tuple(new)

                lax.fori_loop(0, DP, body, regs)

        pltpu.async_copy(src_slice(0), sbufs[0], gsems[0])
        pltpu.async_copy(src_slice(1), sbufs[1], gsems[1])

        @pl.loop(0, T, step=2)
        def _tasks(t0):
            for b in range(2):
                t = t0 + b
                pltpu.make_async_copy(src_slice(t), sbufs[b], gsems[b]).wait()

                @pl.when(t >= 2)
                def _():
                    pltpu.make_async_copy(
                        obufs[b], out_slice(t - 2), ssems[b]).wait()

                compute(t, sbufs[b], obufs[b])

                @pl.when(t + 2 < T)
                def _():
                    pltpu.async_copy(src_slice(t + 2), sbufs[b], gsems[b])

                pltpu.async_copy(obufs[b], out_slice(t), ssems[b])

        pltpu.make_async_copy(obufs[0], out_slice(T - 2), ssems[0]).wait()
        pltpu.make_async_copy(obufs[1], out_slice(T - 1), ssems[1]).wait()

    return gather_k


def kernel(x):
    N, L, D = x.shape
    ids_keep, mask, ids_restore, len_keep = _constants(N, L)
    info = plsc.get_sparse_core_info()
    NC, NS = info.num_cores, info.num_subcores
    xt_flat = jnp.transpose(x, (0, 2, 1)).reshape(N * D * L)
    idx_flat = jnp.asarray(ids_keep.reshape(-1))
    out_flat = _make_gather(N, L, D, len_keep, NC, NS)(xt_flat, idx_flat)
    x_masked = out_flat.reshape(N, D, len_keep).transpose(0, 2, 1)
    return (x_masked, jnp.asarray(mask), jnp.asarray(ids_restore))


# speedup vs baseline: 1.1199x; 1.1199x over previous
import functools

import numpy as np
import jax
import jax.numpy as jnp
from jax import lax
from jax.experimental import pallas as pl
from jax.experimental.pallas import tpu as pltpu
from jax.experimental.pallas import tpu_sc as plsc

_MASK_INDEXES = (1, 4, 7, 10, 13)
_PPI = 64

_consts_cache = {}


def _rotl32(x, r):
    return ((x << np.uint32(r)) | (x >> np.uint32(32 - r))).astype(np.uint32)


def _threefry2x32(k0, k1, x0, x1):
    ks0 = np.uint32(k0)
    ks1 = np.uint32(k1)
    ks2 = np.uint32(ks0 ^ ks1 ^ np.uint32(0x1BD11BDA))
    x0 = (x0 + ks0).astype(np.uint32)
    x1 = (x1 + ks1).astype(np.uint32)
    rots = ((13, 15, 26, 6), (17, 29, 16, 24))
    ks = (ks0, ks1, ks2)
    for i in range(5):
        for r in rots[i % 2]:
            x0 = (x0 + x1).astype(np.uint32)
            x1 = _rotl32(x1, r)
            x1 = (x1 ^ x0).astype(np.uint32)
        x0 = (x0 + ks[(i + 1) % 3]).astype(np.uint32)
        x1 = (x1 + ks[(i + 2) % 3] + np.uint32(i + 1)).astype(np.uint32)
    return x0, x1


def _uniform_key42(shape):
    n = int(np.prod(shape))
    i64 = np.arange(n, dtype=np.uint64)
    c_hi = (i64 >> np.uint64(32)).astype(np.uint32)
    c_lo = (i64 & np.uint64(0xFFFFFFFF)).astype(np.uint32)
    b0, b1 = _threefry2x32(0, 42, c_hi, c_lo)
    bits = (b0 ^ b1).astype(np.uint32)
    f = ((bits >> np.uint32(9)) | np.uint32(0x3F800000)).view(np.float32)
    f = f - np.float32(1.0)
    return np.maximum(np.float32(0.0), f).reshape(shape)


def _constants(N, L):
    ck = (N, L)
    if ck not in _consts_cache:
        noise = _uniform_key42((N, L))
        masked_pos = np.array(
            [idx * _PPI + i for idx in _MASK_INDEXES for i in range(_PPI)],
            dtype=np.int64,
        )
        noise[:, masked_pos] = 2.0
        len_keep = L - masked_pos.size
        ids_shuffle = np.argsort(noise, axis=1, kind="stable").astype(np.int32)
        ids_restore = np.argsort(ids_shuffle, axis=1, kind="stable").astype(np.int32)
        ids_keep = ids_shuffle[:, :len_keep]
        mask = np.ones((N, L), dtype=np.float32)
        mask[:, :len_keep] = 0.0
        mask = np.take_along_axis(mask, ids_restore, axis=1)
        gidx = (ids_keep.astype(np.int64) + np.arange(N, dtype=np.int64)[:, None] * L)
        gidx = gidx.astype(np.int32).reshape(-1)
        _consts_cache[ck] = (gidx, mask, ids_restore, len_keep)
    return _consts_cache[ck]


def _make_gather(num_rows_total, B, D, NC, NS):
    NW = NC * NS
    b_per_w = B // NW
    R = 128
    C = b_per_w // R
    DP = 256
    mesh = plsc.VectorSubcoreMesh(core_axis_name="c", subcore_axis_name="s")

    @functools.partial(
        pl.kernel,
        mesh=mesh,
        compiler_params=pltpu.CompilerParams(use_tc_tiling_on_sc=True),
        out_type=jax.ShapeDtypeStruct((B, DP), jnp.float32),
        scratch_types=[
            pltpu.VMEM((C, R), jnp.int32),
            pltpu.VMEM((R, 128), jnp.float32),
            pltpu.VMEM((R, 128), jnp.float32),
            pltpu.VMEM((R, 128), jnp.float32),
            pltpu.VMEM((R, 128), jnp.float32),
            pltpu.SemaphoreType.DMA,
            pltpu.SemaphoreType.DMA,
            pltpu.SemaphoreType.DMA,
            pltpu.SemaphoreType.DMA,
        ],
    )
    def gather_k(table_hbm, tail_hbm, idx_hbm, out_hbm, idx_v, bufa0, bufa1,
                 bufb0, bufb1, gsem0, gsem1, ssem0, ssem1):
        wid = lax.axis_index("s") * NC + lax.axis_index("c")
        base = wid * b_per_w
        bufsa = (bufa0, bufa1)
        bufsb = (bufb0, bufb1)
        gsems = (gsem0, gsem1)
        ssems = (ssem0, ssem1)
        pltpu.sync_copy(idx_hbm.at[wid], idx_v)

        def start_gathers(c):
            b = c % 2
            ga = pltpu.async_copy(
                table_hbm.at[idx_v.at[c], pl.ds(0, 128)], bufsa[b], gsems[b])
            gb = pltpu.async_copy(
                tail_hbm.at[idx_v.at[c]], bufsb[b], gsems[b])
            return (ga, gb)

        gathers = {}
        stores = {}
        gathers[0] = start_gathers(0)
        for c in range(C):
            b = c % 2
            nxt = c + 1
            if nxt < C:
                if nxt >= 2:
                    for s in stores[nxt - 2]:
                        s.wait()
                gathers[nxt] = start_gathers(nxt)
            for g in gathers[c]:
                g.wait()
            rows = pl.ds(base + c * R, R)
            sa = pltpu.async_copy(
                bufsa[b], out_hbm.at[rows, pl.ds(0, 128)], ssems[b])
            sb = pltpu.async_copy(
                bufsb[b], out_hbm.at[rows, pl.ds(128, 128)], ssems[b])
            stores[c] = (sa, sb)
        for s in stores[C - 2]:
            s.wait()
        for s in stores[C - 1]:
            s.wait()

    return gather_k


def kernel(x):
    N, L, D = x.shape
    gidx, mask, ids_restore, len_keep = _constants(N, L)
    B = N * len_keep
    info = plsc.get_sparse_core_info()
    NC, NS = info.num_cores, info.num_subcores
    x_flat = x.reshape(N * L, D)
    tail = jnp.pad(x[:, :, 128:D].reshape(N * L, D - 128),
                   ((0, 0), (0, 256 - D)))
    idx3 = jnp.asarray(gidx.reshape(NC * NS, -1, 128))
    out = _make_gather(N * L, B, D, NC, NS)(x_flat, tail, idx3)
    x_masked = lax.slice(out, (0, 0), (B, D)).reshape(N, len_keep, D)
    return (x_masked, jnp.asarray(mask), jnp.asarray(ids_restore))


# speedup vs baseline: 1.1737x; 1.0481x over previous
import functools

import numpy as np
import jax
import jax.numpy as jnp
from jax import lax
from jax.experimental import pallas as pl
from jax.experimental.pallas import tpu as pltpu
from jax.experimental.pallas import tpu_sc as plsc

_MASK_INDEXES = (1, 4, 7, 10, 13)
_PPI = 64

_consts_cache = {}


def _rotl32(x, r):
    return ((x << np.uint32(r)) | (x >> np.uint32(32 - r))).astype(np.uint32)


def _threefry2x32(k0, k1, x0, x1):
    ks0 = np.uint32(k0)
    ks1 = np.uint32(k1)
    ks2 = np.uint32(ks0 ^ ks1 ^ np.uint32(0x1BD11BDA))
    x0 = (x0 + ks0).astype(np.uint32)
    x1 = (x1 + ks1).astype(np.uint32)
    rots = ((13, 15, 26, 6), (17, 29, 16, 24))
    ks = (ks0, ks1, ks2)
    for i in range(5):
        for r in rots[i % 2]:
            x0 = (x0 + x1).astype(np.uint32)
            x1 = _rotl32(x1, r)
            x1 = (x1 ^ x0).astype(np.uint32)
        x0 = (x0 + ks[(i + 1) % 3]).astype(np.uint32)
        x1 = (x1 + ks[(i + 2) % 3] + np.uint32(i + 1)).astype(np.uint32)
    return x0, x1


def _uniform_key42(shape):
    n = int(np.prod(shape))
    i64 = np.arange(n, dtype=np.uint64)
    c_hi = (i64 >> np.uint64(32)).astype(np.uint32)
    c_lo = (i64 & np.uint64(0xFFFFFFFF)).astype(np.uint32)
    b0, b1 = _threefry2x32(0, 42, c_hi, c_lo)
    bits = (b0 ^ b1).astype(np.uint32)
    f = ((bits >> np.uint32(9)) | np.uint32(0x3F800000)).view(np.float32)
    f = f - np.float32(1.0)
    return np.maximum(np.float32(0.0), f).reshape(shape)


def _constants(N, L):
    ck = (N, L)
    if ck not in _consts_cache:
        noise = _uniform_key42((N, L))
        masked_pos = np.array(
            [idx * _PPI + i for idx in _MASK_INDEXES for i in range(_PPI)],
            dtype=np.int64,
        )
        noise[:, masked_pos] = 2.0
        len_keep = L - masked_pos.size
        ids_shuffle = np.argsort(noise, axis=1, kind="stable").astype(np.int32)
        ids_restore = np.argsort(ids_shuffle, axis=1, kind="stable").astype(np.int32)
        ids_keep = ids_shuffle[:, :len_keep]
        mask = np.ones((N, L), dtype=np.float32)
        mask[:, :len_keep] = 0.0
        mask = np.take_along_axis(mask, ids_restore, axis=1)
        gidx = (ids_keep.astype(np.int64) + np.arange(N, dtype=np.int64)[:, None] * L)
        gidx = gidx.astype(np.int32).reshape(-1)
        _consts_cache[ck] = (gidx, mask, ids_restore, len_keep)
    return _consts_cache[ck]


def _make_gather(num_rows_total, B, D, NC, NS):
    NW = NC * NS
    b_per_w = B // NW
    R = 128
    C = b_per_w // R
    DP = 256
    mesh = plsc.VectorSubcoreMesh(core_axis_name="c", subcore_axis_name="s")

    @functools.partial(
        pl.kernel,
        mesh=mesh,
        compiler_params=pltpu.CompilerParams(use_tc_tiling_on_sc=True),
        out_type=jax.ShapeDtypeStruct((B, DP), jnp.float32),
        scratch_types=[
            pltpu.VMEM((C, R), jnp.int32),
            pltpu.VMEM((R, DP), jnp.float32),
            pltpu.VMEM((R, DP), jnp.float32),
            pltpu.SemaphoreType.DMA,
            pltpu.SemaphoreType.DMA,
            pltpu.SemaphoreType.DMA,
            pltpu.SemaphoreType.DMA,
        ],
    )
    def gather_k(table_hbm, idx_hbm, out_hbm, idx_v, buf0, buf1,
                 gsem0, gsem1, ssem0, ssem1):
        wid = lax.axis_index("s") * NC + lax.axis_index("c")
        base = wid * b_per_w
        bufs = (buf0, buf1)
        gsems = (gsem0, gsem1)
        ssems = (ssem0, ssem1)
        pltpu.sync_copy(idx_hbm.at[wid], idx_v)
        gathers = {}
        stores = {}
        gathers[0] = pltpu.async_copy(
            table_hbm.at[idx_v.at[0]], bufs[0], gsems[0])
        for c in range(C):
            b = c % 2
            nxt = c + 1
            if nxt < C:
                nb = nxt % 2
                if nxt >= 2:
                    stores[nxt - 2].wait()
                gathers[nxt] = pltpu.async_copy(
                    table_hbm.at[idx_v.at[nxt]], bufs[nb], gsems[nb])
            gathers[c].wait()
            stores[c] = pltpu.async_copy(
                bufs[b], out_hbm.at[pl.ds(base + c * R, R)], ssems[b])
        stores[C - 2].wait()
        stores[C - 1].wait()

    return gather_k


def kernel(x):
    N, L, D = x.shape
    gidx, mask, ids_restore, len_keep = _constants(N, L)
    B = N * len_keep
    info = plsc.get_sparse_core_info()
    NC, NS = info.num_cores, info.num_subcores
    x_flat = x.reshape(N * L, D)
    xp = jnp.pad(x_flat, ((0, 0), (0, 256 - D)))
    idx3 = jnp.asarray(gidx.reshape(NC * NS, -1, 128))
    out = _make_gather(N * L, B, D, NC, NS)(xp, idx3)
    x_masked = lax.slice(out, (0, 0), (B, D)).reshape(N, len_keep, D)
    return (x_masked, jnp.asarray(mask), jnp.asarray(ids_restore))
